# baseline (speedup 1.0000x reference)
import functools

import jax
import jax.numpy as jnp
from jax import lax
from jax.experimental import pallas as pl
from jax.experimental.pallas import tpu as pltpu
from jax.experimental.pallas import tpu_sc as plsc

D = 64
B = 16384
NC, NS = 2, 16
NW = NC * NS
NTC = 7813
PER_W = 245
NBUF = 6
NO = 41
ICH = 2048
SLOTS = 128
FLUSH_AT = SLOTS - 16
OUT_PAD = B + 8
IDCAP = B + 16


@functools.partial(
    pl.kernel,
    mesh=plsc.VectorSubcoreMesh(core_axis_name="c", subcore_axis_name="s"),
    out_type=jax.ShapeDtypeStruct((OUT_PAD, 128), jnp.float32),
    scratch_types=[
        pltpu.VMEM((ICH,), jnp.int32),
        pltpu.VMEM((IDCAP,), jnp.int32),
        pltpu.VMEM((IDCAP,), jnp.int32),
        pltpu.VMEM((SLOTS, 128), jnp.float32),
        pltpu.VMEM((SLOTS,), jnp.int32),
        [pltpu.VMEM((D, 128), jnp.float32)] * NBUF,
        [pltpu.SemaphoreType.DMA] * NBUF,
        pltpu.SemaphoreType.DMA,
    ],
    compiler_params=pltpu.CompilerParams(
        use_tc_tiling_on_sc=True, needs_layout_passes=False
    ),
)
def _sc_gather_stream(tableT, idx_hbm, out_hbm,
                      idx_buf, ids, poss, stage, pos_stage, blks, sems, ssem):
    wid = lax.axis_index("s") * NC + lax.axis_index("c")
    lo = wid * PER_W
    mycnt = jnp.minimum(jnp.int32(PER_W), jnp.int32(NTC) - lo)
    hi = lo + mycnt
    lane_iota = lax.broadcasted_iota(jnp.int32, (16,), 0)

    def chunk(c, cnt):
        pltpu.sync_copy(idx_hbm.at[pl.ds(pl.multiple_of(c * ICH, ICH), ICH)],
                        idx_buf)
        def inner(kk, cnt):
            vec = idx_buf[pl.ds(pl.multiple_of(kk * 16, 16), 16)]
            tc = vec >> 7
            m = (tc >= lo) & (tc < hi)
            plsc.store_compressed(ids.at[pl.ds(cnt, 16)], vec, mask=m)
            posv = c * ICH + kk * 16 + lane_iota
            plsc.store_compressed(poss.at[pl.ds(cnt, 16)], posv, mask=m)
            npop = plsc.all_reduce_population_count(m)
            return cnt + lax.reduce_max(npop, axes=(0,))
        return lax.fori_loop(0, ICH // 16, inner, cnt)

    cnt = lax.fori_loop(0, B // ICH, chunk, jnp.int32(0))
    nck = (cnt + 15) >> 4

    def fire(t, slot):
        coff = jnp.minimum(t, mycnt - 1)
        col = pl.multiple_of((lo + coff) * 128, 128)
        pltpu.async_copy(tableT.at[:, pl.ds(col, 128)], blks[slot], sems[slot])

    def flush(s):
        for k in range(SLOTS // 16):
            slotv = k * 16 + lane_iota
            plsc.store_scatter(pos_stage, [slotv],
                               jnp.full((16,), B, jnp.int32),
                               mask=slotv >= s)
        pltpu.async_copy(stage, out_hbm.at[pos_stage], ssem).wait()
        return jnp.int32(0)

    def drain_process(t, slot, s):
        blk = blks[slot]
        pltpu.make_async_copy(tableT.at[:, pl.ds(0, 128)], blk,
                              sems[slot]).wait()
        col_id = lo + t

        def scan_chunk(ki, s):
            valid = (ki * 16 + lane_iota) < cnt
            vecid = ids[pl.ds(ki * 16, 16)]
            vecpos = poss[pl.ds(ki * 16, 16)]
            m = ((vecid >> 7) == col_id) & valid

            def wcond(st):
                m, s = st
                npop = plsc.all_reduce_population_count(m)
                return lax.reduce_max(npop, axes=(0,)) > 0

            def wbody(st):
                m, s = st
                j = lax.reduce_max(plsc.all_reduce_ffs(m), axes=(0,))
                sel = lane_iota == j
                idv = lax.reduce_max(
                    jnp.where(sel, vecid, jnp.int32(-2147483648)), axes=(0,))
                pv = lax.reduce_max(
                    jnp.where(sel, vecpos, jnp.int32(-2147483648)), axes=(0,))
                lane = idv & 127
                lv = jnp.full((16,), 0, jnp.int32) + lane
                sv = jnp.full((16,), 0, jnp.int32) + s
                for c4 in range(D // 16):
                    row_idx = c4 * 16 + lane_iota
                    vals = plsc.load_gather(blk, [row_idx, lv])
                    plsc.store_scatter(stage, [sv, row_idx], vals)
                plsc.store_scatter(pos_stage, [sv],
                                   jnp.full((16,), 0, jnp.int32) + pv,
                                   mask=lane_iota == 0)
                return m & (~sel), s + 1

            m, s = lax.while_loop(wcond, wbody, (m, s))
            return lax.cond(s > FLUSH_AT, flush, lambda x: x, s)

        return lax.fori_loop(0, nck, scan_chunk, s)

    for b in range(NBUF):
        fire(jnp.int32(b), b)

    def outer(o, s):
        for b in range(NBUF):
            t = o * NBUF + b
            s = drain_process(t, b, s)
            @pl.when(o < NO - 1)
            def _():
                fire(t + NBUF, b)
        return s

    s = lax.fori_loop(0, NO, outer, jnp.int32(0))
    flush(s)


def kernel(data, indices):
    idx = indices.astype(jnp.int32)
    padded = _sc_gather_stream(data.T, idx)
    return padded[:B, :D]
